# baseline (speedup 1.0000x reference)
import functools

import numpy as np

import jax
import jax.numpy as jnp
from jax import lax
from jax.experimental import pallas as pl
from jax.experimental.pallas import tpu as pltpu
from jax.experimental.pallas import tpu_sc as plsc

_NC = 2
_NS = 16
_NW = _NC * _NS
_L = 16

_CH = 128

_BITREV = (0, 8, 4, 12, 2, 10, 6, 14, 1, 9, 5, 13, 3, 11, 7, 15)


def _transpose_tc(Y):
    K, NI = Y.shape
    TW = 512
    grid = (NI + TW - 1) // TW

    def body(y_ref, yt_ref):
        yt_ref[...] = y_ref[...].T

    return pl.pallas_call(
        body,
        grid=(grid,),
        in_specs=[pl.BlockSpec((K, TW), lambda i: (0, i))],
        out_specs=pl.BlockSpec((TW, K), lambda i: (i, 0)),
        out_shape=jax.ShapeDtypeStruct((NI, K), Y.dtype),
    )(Y)


def _sc_score(user_ids, pos_ids, neg_ids, X, YT):
    B = user_ids.shape[0]
    K = X.shape[1]
    assert K == 128
    bpw = B // _NW
    nch = bpw // _CH
    ngrp = _CH // _L

    mesh = plsc.VectorSubcoreMesh(core_axis_name="c", subcore_axis_name="s")

    @functools.partial(
        pl.kernel,
        mesh=mesh,
        out_type=jax.ShapeDtypeStruct((B,), jnp.float32),
        scratch_types=[
            pltpu.VMEM((bpw,), jnp.int32),
            pltpu.VMEM((bpw,), jnp.int32),
            pltpu.VMEM((bpw,), jnp.int32),
            pltpu.VMEM((_CH, 128), jnp.float32),
            pltpu.VMEM((_CH, 128), jnp.float32),
            pltpu.VMEM((_CH, 128), jnp.float32),
            pltpu.VMEM((_CH, 128), jnp.float32),
            pltpu.VMEM((_CH, 128), jnp.float32),
            pltpu.VMEM((_CH, 128), jnp.float32),
            pltpu.VMEM((bpw,), jnp.float32),
            pltpu.SemaphoreType.DMA,
            pltpu.SemaphoreType.DMA,
            pltpu.SemaphoreType.DMA,
        ],
    )
    def k(uid_hbm, pid_hbm, nid_hbm, x_hbm, yt_hbm, out_hbm,
          uix, pix, nix, ua, pa, na, ub2, pb2, nb2, ob,
          sem_i, sem_a, sem_b):
        wid = lax.axis_index("s") * _NC + lax.axis_index("c")
        base = wid * bpw

        c1 = pltpu.async_copy(uid_hbm.at[pl.ds(base, bpw)], uix, sem_i)
        c2 = pltpu.async_copy(pid_hbm.at[pl.ds(base, bpw)], pix, sem_i)
        c3 = pltpu.async_copy(nid_hbm.at[pl.ds(base, bpw)], nix, sem_i)
        c1.wait()
        c2.wait()
        c3.wait()

        bufs = ((ua, pa, na, sem_a), (ub2, pb2, nb2, sem_b))

        def launch(cc, parity):
            u, p, n, sem = bufs[parity]
            off = pl.ds(cc * _CH, _CH)
            pltpu.async_copy(x_hbm.at[uix.at[off]], u, sem)
            pltpu.async_copy(yt_hbm.at[pix.at[off]], p, sem)
            pltpu.async_copy(yt_hbm.at[nix.at[off]], n, sem)

        def drain(parity):
            u, p, n, sem = bufs[parity]
            dummy = x_hbm.at[pl.ds(0, _CH)]
            pltpu.make_async_copy(dummy, u, sem).wait()
            pltpu.make_async_copy(dummy, p, sem).wait()
            pltpu.make_async_copy(dummy, n, sem).wait()

        def compute(c, parity):
            u_buf, p_buf, n_buf, _ = bufs[parity]
            lanes = lax.iota(jnp.int32, _L)

            def pg(x, idx):
                return x.at[idx].get(mode="promise_in_bounds")

            def half(hg, prev):
                odd = (hg % 2) == 1
                half_off = hg % 2

                def leaf(p):
                    r = (hg // 2) * _L + _BITREV[p] + half_off
                    acc = jnp.zeros((_L,), jnp.float32)
                    for k in range(K // _L):
                        u = u_buf[r, pl.ds(k * _L, _L)]
                        p_ = p_buf[r, pl.ds(k * _L, _L)]
                        n_ = n_buf[r, pl.ds(k * _L, _L)]
                        acc = acc + u * (p_ - n_)
                    return acc

                def build(lo, hi, ms):
                    if hi - lo == 1:
                        return leaf(lo)
                    mid = (lo + hi) // 2
                    m = ms[0]
                    x = build(lo, mid, ms[1:])
                    y = build(mid, hi, ms[1:])
                    return jnp.where((lanes & m) == 0,
                                     x + pg(x, lanes ^ m),
                                     y + pg(y, lanes ^ m))

                sub = build(0, 8, (2, 4, 8))

                @pl.when(odd)
                def _():
                    tot = jnp.where((lanes & 1) == 0,
                                    prev + pg(prev, lanes ^ 1),
                                    sub + pg(sub, lanes ^ 1))
                    ob[pl.ds(c * _CH + (hg // 2) * _L, _L)] = tot

                return sub

            lax.fori_loop(0, 2 * ngrp, half,
                          jnp.zeros((_L,), jnp.float32))

        launch(0, 0)

        def chunk_body(c, _):
            even = (c % 2) == 0

            @pl.when(jnp.logical_and(even, c + 1 < nch))
            def _():
                launch(c + 1, 1)

            @pl.when(jnp.logical_and(jnp.logical_not(even), c + 1 < nch))
            def _():
                launch(c + 1, 0)

            @pl.when(even)
            def _():
                drain(0)
                compute(c, 0)

            @pl.when(jnp.logical_not(even))
            def _():
                drain(1)
                compute(c, 1)

            return 0

        lax.fori_loop(0, nch, chunk_body, 0)

        pltpu.sync_copy(ob, out_hbm.at[pl.ds(base, bpw)])

    return k(user_ids, pos_ids, neg_ids, X, YT)


def kernel(user_ids, pos_item_ids, neg_item_ids, X, Y):
    user_ids = user_ids.astype(jnp.int32)
    pos_item_ids = pos_item_ids.astype(jnp.int32)
    neg_item_ids = neg_item_ids.astype(jnp.int32)
    YT = jnp.transpose(Y)
    return _sc_score(user_ids, pos_item_ids, neg_item_ids, X, YT)


# speedup vs baseline: 1.0919x; 1.0919x over previous
import functools

import numpy as np

import jax
import jax.numpy as jnp
from jax import lax
from jax.experimental import pallas as pl
from jax.experimental.pallas import tpu as pltpu
from jax.experimental.pallas import tpu_sc as plsc

_NC = 2
_NS = 16
_NW = _NC * _NS
_L = 16

_CH = 64

_GDN = lax.GatherDimensionNumbers(
    offset_dims=(), collapsed_slice_dims=(0,), start_index_map=(0,))


def _perm(x, idx):
    return lax.gather(x, idx[:, None], _GDN, (1,),
                      mode=lax.GatherScatterMode.PROMISE_IN_BOUNDS)

_BITREV = (0, 8, 4, 12, 2, 10, 6, 14, 1, 9, 5, 13, 3, 11, 7, 15)


def _transpose_tc(Y):
    K, NI = Y.shape
    TW = 512
    grid = (NI + TW - 1) // TW

    def body(y_ref, yt_ref):
        yt_ref[...] = y_ref[...].T

    return pl.pallas_call(
        body,
        grid=(grid,),
        in_specs=[pl.BlockSpec((K, TW), lambda i: (0, i))],
        out_specs=pl.BlockSpec((TW, K), lambda i: (i, 0)),
        out_shape=jax.ShapeDtypeStruct((NI, K), Y.dtype),
    )(Y)


def _sc_score(user_ids, pos_ids, neg_ids, X, YT):
    B = user_ids.shape[0]
    K = X.shape[1]
    assert K == 128
    bpw = B // _NW
    nch = bpw // _CH
    ngrp = _CH // _L

    mesh = plsc.VectorSubcoreMesh(core_axis_name="c", subcore_axis_name="s")

    @functools.partial(
        pl.kernel,
        mesh=mesh,
        out_type=jax.ShapeDtypeStruct((B,), jnp.float32),
        scratch_types=[
            pltpu.VMEM((bpw,), jnp.int32),
            pltpu.VMEM((bpw,), jnp.int32),
            pltpu.VMEM((bpw,), jnp.int32),
            pltpu.VMEM((_CH, 128), jnp.float32),
            pltpu.VMEM((_CH, 128), jnp.float32),
            pltpu.VMEM((_CH, 128), jnp.float32),
            pltpu.VMEM((_CH, 128), jnp.float32),
            pltpu.VMEM((_CH, 128), jnp.float32),
            pltpu.VMEM((_CH, 128), jnp.float32),
            pltpu.VMEM((bpw,), jnp.float32),
            pltpu.SemaphoreType.DMA,
            pltpu.SemaphoreType.DMA,
            pltpu.SemaphoreType.DMA,
        ],
    )
    def k(uid_hbm, pid_hbm, nid_hbm, x_hbm, yt_hbm, out_hbm,
          uix, pix, nix, ua, pa, na, ub2, pb2, nb2, ob,
          sem_i, sem_a, sem_b):
        wid = lax.axis_index("s") * _NC + lax.axis_index("c")
        base = wid * bpw

        c1 = pltpu.async_copy(uid_hbm.at[pl.ds(base, bpw)], uix, sem_i)
        c2 = pltpu.async_copy(pid_hbm.at[pl.ds(base, bpw)], pix, sem_i)
        c3 = pltpu.async_copy(nid_hbm.at[pl.ds(base, bpw)], nix, sem_i)
        c1.wait()
        c2.wait()
        c3.wait()

        bufs = ((ua, pa, na, sem_a), (ub2, pb2, nb2, sem_b))

        def launch(cc, parity):
            u, p, n, sem = bufs[parity]
            off = pl.ds(cc * _CH, _CH)
            pltpu.async_copy(x_hbm.at[uix.at[off]], u, sem)
            pltpu.async_copy(yt_hbm.at[pix.at[off]], p, sem)
            pltpu.async_copy(yt_hbm.at[nix.at[off]], n, sem)

        def drain(parity):
            u, p, n, sem = bufs[parity]
            dummy = x_hbm.at[pl.ds(0, _CH)]
            pltpu.make_async_copy(dummy, u, sem).wait()
            pltpu.make_async_copy(dummy, p, sem).wait()
            pltpu.make_async_copy(dummy, n, sem).wait()

        def compute(c, parity):
            u_buf, p_buf, n_buf, _ = bufs[parity]
            lanes = lax.iota(jnp.int32, _L)

            def half(hg, prev):
                odd = (hg % 2) == 1
                half_off = hg % 2

                def leaf(p):
                    r = (hg // 2) * _L + _BITREV[p] + half_off

                    def chain(ks):
                        acc = None
                        for k in ks:
                            u = u_buf[r, pl.ds(k * _L, _L)]
                            p_ = p_buf[r, pl.ds(k * _L, _L)]
                            n_ = n_buf[r, pl.ds(k * _L, _L)]
                            t = u * (p_ - n_)
                            acc = t if acc is None else acc + t
                        return acc

                    nk = K // _L
                    return chain(range(nk // 2)) + chain(range(nk // 2, nk))

                def build(lo, hi, ms):
                    if hi - lo == 1:
                        return leaf(lo)
                    mid = (lo + hi) // 2
                    m = ms[0]
                    x = build(lo, mid, ms[1:])
                    y = build(mid, hi, ms[1:])
                    return jnp.where((lanes & m) == 0,
                                     x + _perm(x, lanes ^ m),
                                     y + _perm(y, lanes ^ m))

                sub = build(0, 8, (2, 4, 8))

                @pl.when(odd)
                def _():
                    tot = jnp.where((lanes & 1) == 0,
                                    prev + _perm(prev, lanes ^ 1),
                                    sub + _perm(sub, lanes ^ 1))
                    ob[pl.ds(c * _CH + (hg // 2) * _L, _L)] = tot

                return sub

            lax.fori_loop(0, 2 * ngrp, half,
                          jnp.zeros((_L,), jnp.float32))

        launch(0, 0)

        def chunk_body(c, _):
            even = (c % 2) == 0

            @pl.when(jnp.logical_and(even, c + 1 < nch))
            def _():
                launch(c + 1, 1)

            @pl.when(jnp.logical_and(jnp.logical_not(even), c + 1 < nch))
            def _():
                launch(c + 1, 0)

            @pl.when(even)
            def _():
                drain(0)
                compute(c, 0)

            @pl.when(jnp.logical_not(even))
            def _():
                drain(1)
                compute(c, 1)

            return 0

        lax.fori_loop(0, nch, chunk_body, 0)

        pltpu.sync_copy(ob, out_hbm.at[pl.ds(base, bpw)])

    return k(user_ids, pos_ids, neg_ids, X, YT)


def kernel(user_ids, pos_item_ids, neg_item_ids, X, Y):
    user_ids = user_ids.astype(jnp.int32)
    pos_item_ids = pos_item_ids.astype(jnp.int32)
    neg_item_ids = neg_item_ids.astype(jnp.int32)
    YT = jnp.transpose(Y)
    return _sc_score(user_ids, pos_item_ids, neg_item_ids, X, YT)


# speedup vs baseline: 1.3772x; 1.2613x over previous
import functools

import numpy as np

import jax
import jax.numpy as jnp
from jax import lax
from jax.experimental import pallas as pl
from jax.experimental.pallas import tpu as pltpu
from jax.experimental.pallas import tpu_sc as plsc

_NC = 2
_NS = 16
_NW = _NC * _NS
_L = 16

_CH = 64

_GDN = lax.GatherDimensionNumbers(
    offset_dims=(), collapsed_slice_dims=(0,), start_index_map=(0,))


def _perm(x, idx):
    return lax.gather(x, idx[:, None], _GDN, (1,),
                      mode=lax.GatherScatterMode.PROMISE_IN_BOUNDS)

_BITREV = (0, 8, 4, 12, 2, 10, 6, 14, 1, 9, 5, 13, 3, 11, 7, 15)


def _transpose_tc(Y):
    K, NI = Y.shape
    TW = 512
    grid = (NI + TW - 1) // TW

    def body(y_ref, yt_ref):
        yt_ref[...] = y_ref[...].T

    return pl.pallas_call(
        body,
        grid=(grid,),
        in_specs=[pl.BlockSpec((K, TW), lambda i: (0, i))],
        out_specs=pl.BlockSpec((TW, K), lambda i: (i, 0)),
        out_shape=jax.ShapeDtypeStruct((NI, K), Y.dtype),
    )(Y)


def _sc_score(user_ids, pos_ids, neg_ids, X, YT):
    B = user_ids.shape[0]
    K = X.shape[1]
    assert K == 128
    bpw = B // _NW
    nch = bpw // _CH
    ngrp = _CH // _L

    mesh = plsc.VectorSubcoreMesh(core_axis_name="c", subcore_axis_name="s")

    @functools.partial(
        pl.kernel,
        mesh=mesh,
        out_type=jax.ShapeDtypeStruct((B,), jnp.float32),
        scratch_types=[
            pltpu.VMEM((bpw,), jnp.int32),
            pltpu.VMEM((bpw,), jnp.int32),
            pltpu.VMEM((bpw,), jnp.int32),
            pltpu.VMEM((_CH, 128), jnp.float32),
            pltpu.VMEM((_CH, 128), jnp.float32),
            pltpu.VMEM((_CH, 128), jnp.float32),
            pltpu.VMEM((_CH, 128), jnp.float32),
            pltpu.VMEM((_CH, 128), jnp.float32),
            pltpu.VMEM((_CH, 128), jnp.float32),
            pltpu.VMEM((bpw,), jnp.float32),
            pltpu.VMEM((_CH // 2 * _L,), jnp.float32),
            pltpu.SemaphoreType.DMA,
            pltpu.SemaphoreType.DMA,
            pltpu.SemaphoreType.DMA,
        ],
    )
    def k(uid_hbm, pid_hbm, nid_hbm, x_hbm, yt_hbm, out_hbm,
          uix, pix, nix, ua, pa, na, ub2, pb2, nb2, ob, pairbuf,
          sem_i, sem_a, sem_b):
        wid = lax.axis_index("s") * _NC + lax.axis_index("c")
        base = wid * bpw

        c1 = pltpu.async_copy(uid_hbm.at[pl.ds(base, bpw)], uix, sem_i)
        c2 = pltpu.async_copy(pid_hbm.at[pl.ds(base, bpw)], pix, sem_i)
        c3 = pltpu.async_copy(nid_hbm.at[pl.ds(base, bpw)], nix, sem_i)
        c1.wait()
        c2.wait()
        c3.wait()

        bufs = ((ua, pa, na, sem_a), (ub2, pb2, nb2, sem_b))

        def launch(cc, parity):
            u, p, n, sem = bufs[parity]
            off = pl.ds(cc * _CH, _CH)
            pltpu.async_copy(x_hbm.at[uix.at[off]], u, sem)
            pltpu.async_copy(yt_hbm.at[pix.at[off]], p, sem)
            pltpu.async_copy(yt_hbm.at[nix.at[off]], n, sem)

        def drain(parity):
            u, p, n, sem = bufs[parity]
            dummy = x_hbm.at[pl.ds(0, _CH)]
            pltpu.make_async_copy(dummy, u, sem).wait()
            pltpu.make_async_copy(dummy, p, sem).wait()
            pltpu.make_async_copy(dummy, n, sem).wait()

        def compute(c, parity):
            u_buf, p_buf, n_buf, _ = bufs[parity]
            lanes = lax.iota(jnp.int32, _L)

            def merge(x, y, m):
                return jnp.where((lanes & m) == 0,
                                 x + _perm(x, lanes ^ m),
                                 y + _perm(y, lanes ^ m))

            def leaf(r):
                def chain(ks):
                    acc = None
                    for k in ks:
                        u = u_buf[r, pl.ds(k * _L, _L)]
                        p_ = p_buf[r, pl.ds(k * _L, _L)]
                        n_ = n_buf[r, pl.ds(k * _L, _L)]
                        t = u * (p_ - n_)
                        acc = t if acc is None else acc + t
                    return acc

                nk = K // _L
                return chain(range(nk // 2)) + chain(range(nk // 2, nk))

            def pair_body(j, _):
                gg = j // 8
                jj = j % 8
                br = ((jj & 1) << 2) | (jj & 2) | ((jj >> 2) & 1)
                r1 = gg * _L + br
                x = leaf(r1)
                y = leaf(r1 + 8)
                pairbuf[pl.ds(j * _L, _L)] = merge(x, y, 8)
                return 0

            lax.fori_loop(0, _CH // 2, pair_body, 0)

            def group_body(g, _):
                v = [pairbuf[pl.ds((g * 8 + i) * _L, _L)]
                     for i in range(8)]
                n0 = merge(v[0], v[1], 4)
                n1 = merge(v[2], v[3], 4)
                n2 = merge(v[4], v[5], 4)
                n3 = merge(v[6], v[7], 4)
                q0 = merge(n0, n1, 2)
                q1 = merge(n2, n3, 2)
                ob[pl.ds(c * _CH + g * _L, _L)] = merge(q0, q1, 1)
                return 0

            lax.fori_loop(0, ngrp, group_body, 0)

        launch(0, 0)

        def chunk_body(c, _):
            even = (c % 2) == 0

            @pl.when(jnp.logical_and(even, c + 1 < nch))
            def _():
                launch(c + 1, 1)

            @pl.when(jnp.logical_and(jnp.logical_not(even), c + 1 < nch))
            def _():
                launch(c + 1, 0)

            @pl.when(even)
            def _():
                drain(0)
                compute(c, 0)

            @pl.when(jnp.logical_not(even))
            def _():
                drain(1)
                compute(c, 1)

            return 0

        lax.fori_loop(0, nch, chunk_body, 0)

        pltpu.sync_copy(ob, out_hbm.at[pl.ds(base, bpw)])

    return k(user_ids, pos_ids, neg_ids, X, YT)


def kernel(user_ids, pos_item_ids, neg_item_ids, X, Y):
    user_ids = user_ids.astype(jnp.int32)
    pos_item_ids = pos_item_ids.astype(jnp.int32)
    neg_item_ids = neg_item_ids.astype(jnp.int32)
    YT = jnp.transpose(Y)
    return _sc_score(user_ids, pos_item_ids, neg_item_ids, X, YT)


# speedup vs baseline: 1.4094x; 1.0233x over previous
import functools

import numpy as np

import jax
import jax.numpy as jnp
from jax import lax
from jax.experimental import pallas as pl
from jax.experimental.pallas import tpu as pltpu
from jax.experimental.pallas import tpu_sc as plsc

_NC = 2
_NS = 16
_NW = _NC * _NS
_L = 16

_CH = 64

_GDN = lax.GatherDimensionNumbers(
    offset_dims=(), collapsed_slice_dims=(0,), start_index_map=(0,))


def _perm(x, idx):
    return lax.gather(x, idx[:, None], _GDN, (1,),
                      mode=lax.GatherScatterMode.PROMISE_IN_BOUNDS)

_BITREV = (0, 8, 4, 12, 2, 10, 6, 14, 1, 9, 5, 13, 3, 11, 7, 15)


def _transpose_tc(Y):
    K, NI = Y.shape
    TW = 512
    grid = (NI + TW - 1) // TW

    def body(y_ref, yt_ref):
        yt_ref[...] = y_ref[...].T

    return pl.pallas_call(
        body,
        grid=(grid,),
        in_specs=[pl.BlockSpec((K, TW), lambda i: (0, i))],
        out_specs=pl.BlockSpec((TW, K), lambda i: (i, 0)),
        out_shape=jax.ShapeDtypeStruct((NI, K), Y.dtype),
    )(Y)


def _sc_score(user_ids, pos_ids, neg_ids, X, YT):
    B = user_ids.shape[0]
    K = X.shape[1]
    assert K == 128
    bpw = B // _NW
    nch = bpw // _CH
    ngrp = _CH // _L

    mesh = plsc.VectorSubcoreMesh(core_axis_name="c", subcore_axis_name="s")

    @functools.partial(
        pl.kernel,
        mesh=mesh,
        out_type=jax.ShapeDtypeStruct((B,), jnp.float32),
        scratch_types=[
            pltpu.VMEM((bpw,), jnp.int32),
            pltpu.VMEM((bpw,), jnp.int32),
            pltpu.VMEM((bpw,), jnp.int32),
            pltpu.VMEM((_CH, 128), jnp.float32),
            pltpu.VMEM((_CH, 128), jnp.float32),
            pltpu.VMEM((_CH, 128), jnp.float32),
            pltpu.VMEM((_CH, 128), jnp.float32),
            pltpu.VMEM((_CH, 128), jnp.float32),
            pltpu.VMEM((_CH, 128), jnp.float32),
            pltpu.VMEM((bpw,), jnp.float32),
            pltpu.VMEM((_CH // 2 * _L,), jnp.float32),
            pltpu.SemaphoreType.DMA,
            pltpu.SemaphoreType.DMA,
            pltpu.SemaphoreType.DMA,
        ],
    )
    def k(uid_hbm, pid_hbm, nid_hbm, x_hbm, yt_hbm, out_hbm,
          uix, pix, nix, ua, pa, na, ub2, pb2, nb2, ob, pairbuf,
          sem_i, sem_a, sem_b):
        wid = lax.axis_index("s") * _NC + lax.axis_index("c")
        base = wid * bpw

        c1 = pltpu.async_copy(uid_hbm.at[pl.ds(base, bpw)], uix, sem_i)
        c2 = pltpu.async_copy(pid_hbm.at[pl.ds(base, bpw)], pix, sem_i)
        c3 = pltpu.async_copy(nid_hbm.at[pl.ds(base, bpw)], nix, sem_i)
        c1.wait()
        c2.wait()
        c3.wait()

        bufs = ((ua, pa, na, sem_a), (ub2, pb2, nb2, sem_b))

        def launch(cc, parity):
            u, p, n, sem = bufs[parity]
            off = pl.ds(cc * _CH, _CH)
            pltpu.async_copy(x_hbm.at[uix.at[off]], u, sem)
            pltpu.async_copy(yt_hbm.at[pix.at[off]], p, sem)
            pltpu.async_copy(yt_hbm.at[nix.at[off]], n, sem)

        def drain(parity):
            u, p, n, sem = bufs[parity]
            dummy = x_hbm.at[pl.ds(0, _CH)]
            pltpu.make_async_copy(dummy, u, sem).wait()
            pltpu.make_async_copy(dummy, p, sem).wait()
            pltpu.make_async_copy(dummy, n, sem).wait()

        def compute(c, parity):
            u_buf, p_buf, n_buf, _ = bufs[parity]
            lanes = lax.iota(jnp.int32, _L)

            def merge(x, y, m):
                return jnp.where((lanes & m) == 0,
                                 x + _perm(x, lanes ^ m),
                                 y + _perm(y, lanes ^ m))

            def leaf(r):
                def chain(ks):
                    acc = None
                    for k in ks:
                        u = u_buf[r, pl.ds(k * _L, _L)]
                        p_ = p_buf[r, pl.ds(k * _L, _L)]
                        n_ = n_buf[r, pl.ds(k * _L, _L)]
                        t = u * (p_ - n_)
                        acc = t if acc is None else acc + t
                    return acc

                nk = K // _L
                return chain(range(nk // 2)) + chain(range(nk // 2, nk))

            @plsc.parallel_loop(0, _CH // 2, unroll=2)
            def pair_body(j):
                gg = j // 8
                jj = j % 8
                br = ((jj & 1) << 2) | (jj & 2) | ((jj >> 2) & 1)
                r1 = gg * _L + br
                x = leaf(r1)
                y = leaf(r1 + 8)
                pairbuf[pl.ds(j * _L, _L)] = merge(x, y, 8)

            def group_body(g, _):
                v = [pairbuf[pl.ds((g * 8 + i) * _L, _L)]
                     for i in range(8)]
                n0 = merge(v[0], v[1], 4)
                n1 = merge(v[2], v[3], 4)
                n2 = merge(v[4], v[5], 4)
                n3 = merge(v[6], v[7], 4)
                q0 = merge(n0, n1, 2)
                q1 = merge(n2, n3, 2)
                ob[pl.ds(c * _CH + g * _L, _L)] = merge(q0, q1, 1)
                return 0

            lax.fori_loop(0, ngrp, group_body, 0)

        launch(0, 0)

        def chunk_body(c, _):
            even = (c % 2) == 0

            @pl.when(jnp.logical_and(even, c + 1 < nch))
            def _():
                launch(c + 1, 1)

            @pl.when(jnp.logical_and(jnp.logical_not(even), c + 1 < nch))
            def _():
                launch(c + 1, 0)

            @pl.when(even)
            def _():
                drain(0)
                compute(c, 0)

            @pl.when(jnp.logical_not(even))
            def _():
                drain(1)
                compute(c, 1)

            return 0

        lax.fori_loop(0, nch, chunk_body, 0)

        pltpu.sync_copy(ob, out_hbm.at[pl.ds(base, bpw)])

    return k(user_ids, pos_ids, neg_ids, X, YT)


def kernel(user_ids, pos_item_ids, neg_item_ids, X, Y):
    user_ids = user_ids.astype(jnp.int32)
    pos_item_ids = pos_item_ids.astype(jnp.int32)
    neg_item_ids = neg_item_ids.astype(jnp.int32)
    YT = jnp.transpose(Y)
    return _sc_score(user_ids, pos_item_ids, neg_item_ids, X, YT)


# speedup vs baseline: 1.4109x; 1.0011x over previous
import functools

import jax
import jax.numpy as jnp
from jax import lax
from jax.experimental import pallas as pl
from jax.experimental.pallas import tpu as pltpu
from jax.experimental.pallas import tpu_sc as plsc

_NC = 2
_NS = 16
_NW = _NC * _NS
_L = 16

_CH = 64

_GDN = lax.GatherDimensionNumbers(
    offset_dims=(), collapsed_slice_dims=(0,), start_index_map=(0,))


def _perm(x, idx):
    return lax.gather(x, idx[:, None], _GDN, (1,),
                      mode=lax.GatherScatterMode.PROMISE_IN_BOUNDS)

def _sc_score(user_ids, pos_ids, neg_ids, X, YT):
    B = user_ids.shape[0]
    K = X.shape[1]
    assert K == 128
    bpw = B // _NW
    nch = bpw // _CH
    ngrp = _CH // _L

    mesh = plsc.VectorSubcoreMesh(core_axis_name="c", subcore_axis_name="s")

    @functools.partial(
        pl.kernel,
        mesh=mesh,
        out_type=jax.ShapeDtypeStruct((B,), jnp.float32),
        scratch_types=[
            pltpu.VMEM((bpw,), jnp.int32),
            pltpu.VMEM((bpw,), jnp.int32),
            pltpu.VMEM((bpw,), jnp.int32),
            pltpu.VMEM((_CH, 128), jnp.float32),
            pltpu.VMEM((_CH, 128), jnp.float32),
            pltpu.VMEM((_CH, 128), jnp.float32),
            pltpu.VMEM((_CH, 128), jnp.float32),
            pltpu.VMEM((_CH, 128), jnp.float32),
            pltpu.VMEM((_CH, 128), jnp.float32),
            pltpu.VMEM((bpw,), jnp.float32),
            pltpu.VMEM((_CH // 2 * _L,), jnp.float32),
            pltpu.SemaphoreType.DMA,
            pltpu.SemaphoreType.DMA,
            pltpu.SemaphoreType.DMA,
        ],
    )
    def k(uid_hbm, pid_hbm, nid_hbm, x_hbm, yt_hbm, out_hbm,
          uix, pix, nix, ua, pa, na, ub2, pb2, nb2, ob, pairbuf,
          sem_i, sem_a, sem_b):
        wid = lax.axis_index("s") * _NC + lax.axis_index("c")
        base = wid * bpw

        c1 = pltpu.async_copy(uid_hbm.at[pl.ds(base, bpw)], uix, sem_i)
        c2 = pltpu.async_copy(pid_hbm.at[pl.ds(base, bpw)], pix, sem_i)
        c3 = pltpu.async_copy(nid_hbm.at[pl.ds(base, bpw)], nix, sem_i)
        c1.wait()
        c2.wait()
        c3.wait()

        bufs = ((ua, pa, na, sem_a), (ub2, pb2, nb2, sem_b))

        def launch(cc, parity):
            u, p, n, sem = bufs[parity]
            off = pl.ds(cc * _CH, _CH)
            pltpu.async_copy(x_hbm.at[uix.at[off]], u, sem)
            pltpu.async_copy(yt_hbm.at[pix.at[off]], p, sem)
            pltpu.async_copy(yt_hbm.at[nix.at[off]], n, sem)

        def drain(parity):
            u, p, n, sem = bufs[parity]
            dummy = x_hbm.at[pl.ds(0, _CH)]
            pltpu.make_async_copy(dummy, u, sem).wait()
            pltpu.make_async_copy(dummy, p, sem).wait()
            pltpu.make_async_copy(dummy, n, sem).wait()

        def compute(c, parity):
            u_buf, p_buf, n_buf, _ = bufs[parity]
            lanes = lax.iota(jnp.int32, _L)

            def merge(x, y, m):
                return jnp.where((lanes & m) == 0,
                                 x + _perm(x, lanes ^ m),
                                 y + _perm(y, lanes ^ m))

            def leaf(r):
                def chain(ks):
                    acc = None
                    for k in ks:
                        u = u_buf[r, pl.ds(k * _L, _L)]
                        p_ = p_buf[r, pl.ds(k * _L, _L)]
                        n_ = n_buf[r, pl.ds(k * _L, _L)]
                        t = u * (p_ - n_)
                        acc = t if acc is None else acc + t
                    return acc

                nk = K // _L
                return chain(range(nk // 2)) + chain(range(nk // 2, nk))

            @plsc.parallel_loop(0, _CH // 2, unroll=2)
            def pair_body(j):
                gg = j // 8
                jj = j % 8
                br = ((jj & 1) << 2) | (jj & 2) | ((jj >> 2) & 1)
                r1 = gg * _L + br
                x = leaf(r1)
                y = leaf(r1 + 8)
                pairbuf[pl.ds(j * _L, _L)] = merge(x, y, 8)

            def group_body(g, _):
                v = [pairbuf[pl.ds((g * 8 + i) * _L, _L)]
                     for i in range(8)]
                n0 = merge(v[0], v[1], 4)
                n1 = merge(v[2], v[3], 4)
                n2 = merge(v[4], v[5], 4)
                n3 = merge(v[6], v[7], 4)
                q0 = merge(n0, n1, 2)
                q1 = merge(n2, n3, 2)
                ob[pl.ds(c * _CH + g * _L, _L)] = merge(q0, q1, 1)
                return 0

            lax.fori_loop(0, ngrp, group_body, 0)

        launch(0, 0)

        def chunk_body(c, _):
            even = (c % 2) == 0

            @pl.when(jnp.logical_and(even, c + 1 < nch))
            def _():
                launch(c + 1, 1)

            @pl.when(jnp.logical_and(jnp.logical_not(even), c + 1 < nch))
            def _():
                launch(c + 1, 0)

            @pl.when(even)
            def _():
                drain(0)
                compute(c, 0)

            @pl.when(jnp.logical_not(even))
            def _():
                drain(1)
                compute(c, 1)

            return 0

        lax.fori_loop(0, nch, chunk_body, 0)

        pltpu.sync_copy(ob, out_hbm.at[pl.ds(base, bpw)])

    return k(user_ids, pos_ids, neg_ids, X, YT)


def kernel(user_ids, pos_item_ids, neg_item_ids, X, Y):
    user_ids = user_ids.astype(jnp.int32)
    pos_item_ids = pos_item_ids.astype(jnp.int32)
    neg_item_ids = neg_item_ids.astype(jnp.int32)
    YT = jnp.transpose(Y)
    return _sc_score(user_ids, pos_item_ids, neg_item_ids, X, YT)
